# Initial kernel scaffold; baseline (speedup 1.0000x reference)
import jax
import jax.numpy as jnp
from jax.experimental import pallas as pl


def kernel(seq_len, bias_table):
    raise NotImplementedError("write your pallas kernel here")


# speedup vs baseline: 41.3634x; 41.3634x over previous
import functools

import jax
import jax.numpy as jnp
from jax import lax
from jax.experimental import pallas as pl
from jax.experimental.pallas import tpu as pltpu
from jax.experimental.pallas import tpu_sc as plsc

_NC = 2
_NS = 16
_NW = _NC * _NS


@functools.lru_cache(maxsize=None)
def _make_circulant_kernel(H, S):
  rows_per_w = S // _NW
  W = rows_per_w + S - 8
  mesh = plsc.VectorSubcoreMesh(core_axis_name="c", subcore_axis_name="s")

  @functools.partial(
      pl.kernel,
      mesh=mesh,
      out_type=jax.ShapeDtypeStruct((H, S, S), jnp.float32),
      scratch_types=[
          pltpu.VMEM((8, W), jnp.float32),
          pltpu.SemaphoreType.DMA,
      ],
      compiler_params=pltpu.CompilerParams(use_tc_tiling_on_sc=False),
  )
  def k(e8_hbm, out_hbm, e8_vmem, sem):
    wid = lax.axis_index("s") * _NC + lax.axis_index("c")
    i0 = wid * rows_per_w
    w0 = pl.multiple_of(S - rows_per_w - i0, 8)
    for h in range(H):
      for m in range(8):
        pltpu.sync_copy(e8_hbm.at[h, m, pl.ds(w0, W)], e8_vmem.at[m])

      def fire(r, carry):
        i = i0 + r
        start = (S - 1) - i
        m = lax.rem(start, 8)
        a_loc = pl.multiple_of(start - m - w0, 8)
        pltpu.async_copy(
            e8_vmem.at[m, pl.ds(a_loc, S)], out_hbm.at[h, i], sem)
        return carry

      lax.fori_loop(0, rows_per_w, fire, 0)

      def drain(r, carry):
        pltpu.make_async_copy(
            e8_vmem.at[0, pl.ds(0, S)], out_hbm.at[h, i0], sem).wait()
        return carry

      lax.fori_loop(0, rows_per_w, drain, 0)

  return k


def kernel(seq_len, bias_table):
  del seq_len
  S, H = bias_table.shape
  r = jnp.flip(bias_table, axis=0)
  big = jnp.concatenate([r, r, r], axis=0)
  e8 = jnp.stack([big[m:m + 2 * S] for m in range(8)], axis=0)
  e8 = jnp.transpose(e8, (2, 0, 1))
  return _make_circulant_kernel(H, S)(e8)


# speedup vs baseline: 48.2675x; 1.1669x over previous
import functools

import jax
import jax.numpy as jnp
from jax import lax
from jax.experimental import pallas as pl
from jax.experimental.pallas import tpu as pltpu
from jax.experimental.pallas import tpu_sc as plsc

_NC = 2
_NS = 16
_NW = _NC * _NS


@functools.lru_cache(maxsize=None)
def _make_circulant_kernel(H, S):
  rows_per_w = S // _NW
  n_groups = rows_per_w // 8
  W = rows_per_w + S - 8
  mesh = plsc.VectorSubcoreMesh(core_axis_name="c", subcore_axis_name="s")

  @functools.partial(
      pl.kernel,
      mesh=mesh,
      out_type=jax.ShapeDtypeStruct((H, S, S), jnp.float32),
      scratch_types=[
          pltpu.VMEM((8, W), jnp.float32),
          pltpu.VMEM((8, W), jnp.float32),
          pltpu.SemaphoreType.DMA,
          pltpu.SemaphoreType.DMA,
          pltpu.SemaphoreType.DMA,
      ],
      compiler_params=pltpu.CompilerParams(use_tc_tiling_on_sc=False),
  )
  def k(e8_hbm, out_hbm, buf0, buf1, sem_in, sem_a, sem_b):
    bufs = (buf0, buf1)
    sems = (sem_a, sem_b)
    wid = lax.axis_index("s") * _NC + lax.axis_index("c")
    i0 = pl.multiple_of(wid * rows_per_w, 8)
    w0 = pl.multiple_of(S - rows_per_w - i0, 8)

    def load(h, buf):
      pltpu.async_copy(e8_hbm.at[h, :, pl.ds(w0, W)], buf, sem_in)

    def wait_load(buf):
      pltpu.make_async_copy(e8_hbm.at[0, :, pl.ds(0, W)], buf, sem_in).wait()

    load(0, bufs[0])
    for h in range(H):
      buf, sem = bufs[h % 2], sems[h % 2]
      wait_load(buf)
      for g in range(n_groups):
        a_loc = rows_per_w - 8 - 8 * g
        i_base = i0 + 8 * g
        pltpu.async_copy(
            buf.at[:, pl.ds(a_loc, S)], out_hbm.at[h, pl.ds(i_base, 8)], sem)
      if h >= 1:
        pbuf, psem = bufs[(h - 1) % 2], sems[(h - 1) % 2]
        for g in range(n_groups):
          pltpu.make_async_copy(
              pbuf.at[:, pl.ds(0, S)],
              out_hbm.at[h - 1, pl.ds(i0, 8)], psem).wait()
      if h + 1 < H:
        load(h + 1, bufs[(h + 1) % 2])
    lbuf, lsem = bufs[(H - 1) % 2], sems[(H - 1) % 2]
    for g in range(n_groups):
      pltpu.make_async_copy(
          lbuf.at[:, pl.ds(0, S)], out_hbm.at[H - 1, pl.ds(i0, 8)], lsem).wait()

  return k


def kernel(seq_len, bias_table):
  del seq_len
  S, H = bias_table.shape
  r = jnp.flip(bias_table, axis=0)
  big = jnp.concatenate([r, r, r], axis=0)
  e8r = jnp.stack([big[7 - u:7 - u + 2 * S] for u in range(8)], axis=0)
  e8r = jnp.transpose(e8r, (2, 0, 1))
  return _make_circulant_kernel(H, S)(e8r)
